# Initial kernel scaffold; baseline (speedup 1.0000x reference)
import jax
import jax.numpy as jnp
from jax.experimental import pallas as pl


def kernel(x, edge_index, W1, b1, W2, b2, W3, b3):
    raise NotImplementedError("write your pallas kernel here")


# speedup vs baseline: 7.4783x; 7.4783x over previous
import functools

import jax
import jax.numpy as jnp
from jax import lax
from jax.experimental import pallas as pl
from jax.experimental.pallas import tpu as pltpu
from jax.experimental.pallas import tpu_sc as plsc

N = 10000
E = 160000
D = 256
H = 128
NC, NS = 2, 16
NW = NC * NS
EPT = E // NW
K = 128
NCHUNK = 40
EPT_PAD = NCHUNK * K
NPAD = 10240
RPT = NPAD // NS
ZR = RPT // 4
DUMMY = N
BLK = 1000

_mesh = plsc.VectorSubcoreMesh(
    core_axis_name="c", subcore_axis_name="s", num_cores=NC, num_subcores=NS
)


def _remap_body(r_ref, c_ref, rlo_ref, rhi_ref, co_ref):
    rr = r_ref[...]
    cc = c_ref[...]
    is_self = rr == cc
    co = jnp.where(is_self, DUMMY, cc)
    pad = ((0, 0), (0, 0), (0, EPT_PAD - EPT))
    rp = jnp.pad(rr, pad, constant_values=0)
    rlo_ref[...] = rp
    rhi_ref[...] = rp + N
    co_ref[...] = jnp.pad(co, pad, constant_values=DUMMY)


def _tc_remap(r_arr, c_arr):
    return pl.pallas_call(
        _remap_body,
        grid=(NW,),
        in_specs=[
            pl.BlockSpec((1, 1, EPT), lambda w: (w, 0, 0)),
            pl.BlockSpec((1, 1, EPT), lambda w: (w, 0, 0)),
        ],
        out_specs=[
            pl.BlockSpec((1, 1, EPT_PAD), lambda w: (w, 0, 0)),
            pl.BlockSpec((1, 1, EPT_PAD), lambda w: (w, 0, 0)),
            pl.BlockSpec((1, 1, EPT_PAD), lambda w: (w, 0, 0)),
        ],
        out_shape=[
            jax.ShapeDtypeStruct((NW, 1, EPT_PAD), jnp.int32),
            jax.ShapeDtypeStruct((NW, 1, EPT_PAD), jnp.int32),
            jax.ShapeDtypeStruct((NW, 1, EPT_PAD), jnp.int32),
        ],
    )(r_arr, c_arr)


@functools.partial(
    pl.kernel,
    out_type=jax.ShapeDtypeStruct((NC * NPAD,), jnp.float32),
    mesh=_mesh,
    scratch_types=[
        pltpu.VMEM((NCHUNK, K), jnp.int32),
        pltpu.VMEM((K,), jnp.float32),
        pltpu.VMEM((RPT,), jnp.float32),
        pltpu.VMEM_SHARED((NPAD,), jnp.float32),
    ],
)
def _sc_deg(cidx_hbm, deg_out, cv, ones_v, zb, deg_sp):
    cid = lax.axis_index("c")
    sid = lax.axis_index("s")
    wid = cid * NS + sid
    one16 = jnp.full((16,), 1.0, jnp.float32)
    zero16 = jnp.zeros((16,), jnp.float32)
    for i in range(K // 16):
        ones_v[pl.ds(i * 16, 16)] = one16

    def zrow(i, _):
        zb[pl.ds(i * 16, 16)] = zero16
        return 0

    lax.fori_loop(0, RPT // 16, zrow, 0)
    pltpu.sync_copy(zb, deg_sp.at[pl.ds(sid * RPT, RPT)])
    plsc.subcore_barrier()
    pltpu.sync_copy(cidx_hbm.at[wid], cv)

    def body(j, _):
        pltpu.sync_copy(ones_v, deg_sp.at[cv.at[j]], add=True)
        return 0

    lax.fori_loop(0, NCHUNK, body, 0)
    plsc.subcore_barrier()
    pltpu.sync_copy(deg_sp.at[pl.ds(sid * RPT, RPT)], zb)
    pltpu.sync_copy(zb, deg_out.at[pl.ds(cid * NPAD + sid * RPT, RPT)])


@functools.partial(
    pl.kernel,
    out_type=jax.ShapeDtypeStruct((NC, NPAD, H), jnp.float32),
    mesh=_mesh,
    scratch_types=[
        pltpu.VMEM((NCHUNK, K), jnp.int32),
        pltpu.VMEM((NCHUNK, K), jnp.int32),
        pltpu.VMEM((K, H), jnp.float32),
        pltpu.VMEM((ZR, H), jnp.float32),
        pltpu.VMEM_SHARED((NPAD, H), jnp.float32),
        pltpu.SemaphoreType.DMA,
    ],
)
def _sc_agg(z_hbm, ridx_hbm, cidx_hbm, s_hbm, rv, cv, gbuf, zbuf, acc, sem):
    cid = lax.axis_index("c")
    sid = lax.axis_index("s")
    zero16 = jnp.zeros((16,), jnp.float32)

    def zrow(i, _):
        for k in range(H // 16):
            zbuf[i, pl.ds(k * 16, 16)] = zero16
        return 0

    lax.fori_loop(0, ZR, zrow, 0)
    for q in range(RPT // ZR):
        pltpu.sync_copy(zbuf, acc.at[pl.ds(sid * RPT + q * ZR, ZR)])
    plsc.subcore_barrier()
    for slab_off in (0, NS):
        slab = sid + slab_off
        pltpu.sync_copy(ridx_hbm.at[cid, slab], rv)
        pltpu.sync_copy(cidx_hbm.at[slab], cv)

        def body(j, _):
            pltpu.async_copy(z_hbm.at[rv.at[j]], gbuf, sem).wait()
            pltpu.sync_copy(gbuf, acc.at[cv.at[j]], add=True)
            return 0

        lax.fori_loop(0, NCHUNK, body, 0)
    plsc.subcore_barrier()
    for q in range(RPT // ZR):
        pltpu.sync_copy(acc.at[pl.ds(sid * RPT + q * ZR, ZR)], zbuf)
        pltpu.sync_copy(zbuf, s_hbm.at[cid, pl.ds(sid * RPT + q * ZR, ZR)])


def _dinv(dp_ref):
    deg = dp_ref[:, 0] + dp_ref[:, 1] + 1.0
    return lax.rsqrt(deg)


def _k1_body(x_ref, w_ref, dp_ref, z_ref):
    dinv = _dinv(dp_ref)
    y = jnp.dot(x_ref[...], w_ref[...], preferred_element_type=jnp.float32)
    z = y * dinv[:, None]
    z_ref[0] = z[:, :H]
    z_ref[1] = z[:, H:]


def _mid_body(s_ref, z_ref, dp_ref, w_ref, b_ref, zo_ref):
    dinv = _dinv(dp_ref)
    t = jnp.concatenate([s_ref[0] + z_ref[0], s_ref[1] + z_ref[1]], axis=1)
    h = jnp.maximum(t * dinv[:, None] + b_ref[...], 0.0)
    y = jnp.dot(h, w_ref[...], preferred_element_type=jnp.float32)
    zn = y * dinv[:, None]
    zo_ref[0] = zn[:, :H]
    zo_ref[1] = zn[:, H:]


def _fin_body(s_ref, z_ref, dp_ref, b_ref, o_ref):
    dinv = _dinv(dp_ref)
    t = jnp.concatenate([s_ref[0] + z_ref[0], s_ref[1] + z_ref[1]], axis=1)
    o_ref[...] = t * dinv[:, None] + b_ref[...]


_spec_z = pl.BlockSpec((NC, BLK, H), lambda i: (0, i, 0))
_spec_dp = pl.BlockSpec((BLK, NC), lambda i: (i, 0))
_spec_w = pl.BlockSpec((D, D), lambda i: (0, 0))
_spec_b = pl.BlockSpec((1, D), lambda i: (0, 0))


def _tc_k1(x, W, deg_part):
    return pl.pallas_call(
        _k1_body,
        grid=(N // BLK,),
        in_specs=[
            pl.BlockSpec((BLK, D), lambda i: (i, 0)),
            _spec_w,
            _spec_dp,
        ],
        out_specs=_spec_z,
        out_shape=jax.ShapeDtypeStruct((NC, N, H), jnp.float32),
    )(x, W, deg_part)


def _tc_mid(s, z, deg_part, W, b):
    return pl.pallas_call(
        _mid_body,
        grid=(N // BLK,),
        in_specs=[_spec_z, _spec_z, _spec_dp, _spec_w, _spec_b],
        out_specs=_spec_z,
        out_shape=jax.ShapeDtypeStruct((NC, N, H), jnp.float32),
    )(s, z, deg_part, W, b)


def _tc_fin(s, z, deg_part, b):
    return pl.pallas_call(
        _fin_body,
        grid=(N // BLK,),
        in_specs=[_spec_z, _spec_z, _spec_dp, _spec_b],
        out_specs=pl.BlockSpec((BLK, D), lambda i: (i, 0)),
        out_shape=jax.ShapeDtypeStruct((N, D), jnp.float32),
    )(s, z, deg_part, b)


def kernel(x, edge_index, W1, b1, W2, b2, W3, b3):
    r_arr = edge_index[0].reshape(NW, 1, EPT)
    c_arr = edge_index[1].reshape(NW, 1, EPT)
    rlo, rhi, co = _tc_remap(r_arr, c_arr)
    ridx = jnp.stack([rlo, rhi]).reshape(NC, NW, NCHUNK, K)
    cidx = co.reshape(NW, NCHUNK, K)

    deg_part = _sc_deg(cidx).reshape(NC, NPAD).T

    b1r, b2r, b3r = (b.reshape(1, D) for b in (b1, b2, b3))

    z1 = _tc_k1(x, W1, deg_part)
    s1 = _sc_agg(z1.reshape(NC * N, H), ridx, cidx)
    z2 = _tc_mid(s1, z1, deg_part, W2, b1r)
    s2 = _sc_agg(z2.reshape(NC * N, H), ridx, cidx)
    z3 = _tc_mid(s2, z2, deg_part, W3, b2r)
    s3 = _sc_agg(z3.reshape(NC * N, H), ridx, cidx)
    return _tc_fin(s3, z3, deg_part, b3r)


# speedup vs baseline: 8.2300x; 1.1005x over previous
import functools

import jax
import jax.numpy as jnp
from jax import lax
from jax.experimental import pallas as pl
from jax.experimental.pallas import tpu as pltpu
from jax.experimental.pallas import tpu_sc as plsc

N = 10000
E = 160000
D = 256
H = 128
NC, NS = 2, 16
NW = NC * NS
EPT = E // NW
K = 128
NCHUNK = 40
EPT_PAD = NCHUNK * K
NPAD = 10240
RPT = NPAD // NS
NBUF = 2
DUMMY = N
BLK = 1000

_mesh = plsc.VectorSubcoreMesh(
    core_axis_name="c", subcore_axis_name="s", num_cores=NC, num_subcores=NS
)


def _remap_body(r_ref, c_ref, rlo_ref, rhi_ref, co_ref):
    rr = r_ref[...]
    cc = c_ref[...]
    is_self = rr == cc
    co = jnp.where(is_self, DUMMY, cc)
    pad = ((0, 0), (0, 0), (0, EPT_PAD - EPT))
    rp = jnp.pad(rr, pad, constant_values=0)
    rlo_ref[...] = rp
    rhi_ref[...] = rp + N
    co_ref[...] = jnp.pad(co, pad, constant_values=DUMMY)


def _tc_remap(r_arr, c_arr):
    return pl.pallas_call(
        _remap_body,
        grid=(NW,),
        in_specs=[
            pl.BlockSpec((1, 1, EPT), lambda w: (w, 0, 0)),
            pl.BlockSpec((1, 1, EPT), lambda w: (w, 0, 0)),
        ],
        out_specs=[
            pl.BlockSpec((1, 1, EPT_PAD), lambda w: (w, 0, 0)),
            pl.BlockSpec((1, 1, EPT_PAD), lambda w: (w, 0, 0)),
            pl.BlockSpec((1, 1, EPT_PAD), lambda w: (w, 0, 0)),
        ],
        out_shape=[
            jax.ShapeDtypeStruct((NW, 1, EPT_PAD), jnp.int32),
            jax.ShapeDtypeStruct((NW, 1, EPT_PAD), jnp.int32),
            jax.ShapeDtypeStruct((NW, 1, EPT_PAD), jnp.int32),
        ],
    )(r_arr, c_arr)


@functools.partial(
    pl.kernel,
    out_type=jax.ShapeDtypeStruct((NC * NPAD,), jnp.float32),
    mesh=_mesh,
    scratch_types=[
        pltpu.VMEM((NCHUNK, K), jnp.int32),
        pltpu.VMEM((K,), jnp.float32),
        pltpu.VMEM((RPT,), jnp.float32),
        pltpu.VMEM_SHARED((NPAD,), jnp.float32),
    ],
)
def _sc_deg(cidx_hbm, deg_out, cv, ones_v, zb, deg_sp):
    cid = lax.axis_index("c")
    sid = lax.axis_index("s")
    wid = cid * NS + sid
    one16 = jnp.full((16,), 1.0, jnp.float32)
    zero16 = jnp.zeros((16,), jnp.float32)
    for i in range(K // 16):
        ones_v[pl.ds(i * 16, 16)] = one16

    def zrow(i, _):
        zb[pl.ds(i * 16, 16)] = zero16
        return 0

    lax.fori_loop(0, RPT // 16, zrow, 0)
    pltpu.sync_copy(zb, deg_sp.at[pl.ds(sid * RPT, RPT)])
    plsc.subcore_barrier()
    pltpu.sync_copy(cidx_hbm.at[wid], cv)

    def body(j, _):
        pltpu.sync_copy(ones_v, deg_sp.at[cv.at[j]], add=True)
        return 0

    lax.fori_loop(0, NCHUNK, body, 0)
    plsc.subcore_barrier()
    pltpu.sync_copy(deg_sp.at[pl.ds(sid * RPT, RPT)], zb)
    pltpu.sync_copy(zb, deg_out.at[pl.ds(cid * NPAD + sid * RPT, RPT)])


@functools.partial(
    pl.kernel,
    out_type=jax.ShapeDtypeStruct((NC, NPAD, H), jnp.float32),
    mesh=_mesh,
    scratch_types=[
        pltpu.VMEM((NCHUNK, K), jnp.int32),
        pltpu.VMEM((NCHUNK, K), jnp.int32),
        pltpu.VMEM((K, H), jnp.float32),
        pltpu.VMEM((K, H), jnp.float32),
        pltpu.VMEM_SHARED((NPAD, H), jnp.float32),
        pltpu.SemaphoreType.DMA,
        pltpu.SemaphoreType.DMA,
        pltpu.SemaphoreType.DMA,
        pltpu.SemaphoreType.DMA,
    ],
)
def _sc_agg(z_hbm, ridx_hbm, cidx_hbm, s_hbm, rv, cv, g0, g1,
            acc, gs0, gs1, ss0, ss1):
    cid = lax.axis_index("c")
    sid = lax.axis_index("s")
    gb = (g0, g1)
    gs = (gs0, gs1)
    ss = (ss0, ss1)
    zero16 = jnp.zeros((16,), jnp.float32)

    def zrow(i, _):
        for k in range(H // 16):
            g0[i, pl.ds(k * 16, 16)] = zero16
        return 0

    lax.fori_loop(0, K, zrow, 0)
    zdesc = [
        pltpu.async_copy(g0, acc.at[pl.ds(sid * RPT + q * K, K)], ss0)
        for q in range(RPT // K)
    ]
    for d in zdesc:
        d.wait()
    plsc.subcore_barrier()

    for slab_off in (0, NS):
        slab = sid + slab_off
        pltpu.sync_copy(ridx_hbm.at[cid, slab], rv)
        pltpu.sync_copy(cidx_hbm.at[slab], cv)
        for b in range(NBUF):
            pltpu.async_copy(z_hbm.at[rv.at[b]], gb[b], gs[b])

        def body(t, _):
            for b in range(NBUF):
                j = t * NBUF + b
                pltpu.make_async_copy(z_hbm.at[pl.ds(0, K)], gb[b], gs[b]).wait()
                pltpu.async_copy(gb[b], acc.at[cv.at[j]], ss[b], add=True)
            for b in range(NBUF):
                jn = (t + 1) * NBUF + b
                pltpu.make_async_copy(z_hbm.at[pl.ds(0, K)], gb[b], ss[b]).wait()

                @pl.when(jn < NCHUNK)
                def _():
                    pltpu.async_copy(z_hbm.at[rv.at[jn]], gb[b], gs[b])

            return 0

        lax.fori_loop(0, NCHUNK // NBUF, body, 0)
    plsc.subcore_barrier()
    wdesc = [None, None]
    for q in range(RPT // K):
        b = q % 2
        if wdesc[b] is not None:
            wdesc[b].wait()
        pltpu.sync_copy(acc.at[pl.ds(sid * RPT + q * K, K)], gb[b])
        wdesc[b] = pltpu.async_copy(
            gb[b], s_hbm.at[cid, pl.ds(sid * RPT + q * K, K)], gs[b]
        )
    for d in wdesc:
        if d is not None:
            d.wait()


def _dinv(dp_ref):
    deg = dp_ref[:, 0] + dp_ref[:, 1] + 1.0
    return lax.rsqrt(deg)


def _k1_body(x_ref, w_ref, dp_ref, z_ref):
    dinv = _dinv(dp_ref)
    y = jnp.dot(x_ref[...], w_ref[...], preferred_element_type=jnp.float32)
    z = y * dinv[:, None]
    z_ref[0] = z[:, :H]
    z_ref[1] = z[:, H:]


def _mid_body(s_ref, z_ref, dp_ref, w_ref, b_ref, zo_ref):
    dinv = _dinv(dp_ref)
    t = jnp.concatenate([s_ref[0] + z_ref[0], s_ref[1] + z_ref[1]], axis=1)
    h = jnp.maximum(t * dinv[:, None] + b_ref[...], 0.0)
    y = jnp.dot(h, w_ref[...], preferred_element_type=jnp.float32)
    zn = y * dinv[:, None]
    zo_ref[0] = zn[:, :H]
    zo_ref[1] = zn[:, H:]


def _fin_body(s_ref, z_ref, dp_ref, b_ref, o_ref):
    dinv = _dinv(dp_ref)
    t = jnp.concatenate([s_ref[0] + z_ref[0], s_ref[1] + z_ref[1]], axis=1)
    o_ref[...] = t * dinv[:, None] + b_ref[...]


_spec_z = pl.BlockSpec((NC, BLK, H), lambda i: (0, i, 0))
_spec_dp = pl.BlockSpec((BLK, NC), lambda i: (i, 0))
_spec_w = pl.BlockSpec((D, D), lambda i: (0, 0))
_spec_b = pl.BlockSpec((1, D), lambda i: (0, 0))


def _tc_k1(x, W, deg_part):
    return pl.pallas_call(
        _k1_body,
        grid=(N // BLK,),
        in_specs=[
            pl.BlockSpec((BLK, D), lambda i: (i, 0)),
            _spec_w,
            _spec_dp,
        ],
        out_specs=_spec_z,
        out_shape=jax.ShapeDtypeStruct((NC, N, H), jnp.float32),
    )(x, W, deg_part)


def _tc_mid(s, z, deg_part, W, b):
    return pl.pallas_call(
        _mid_body,
        grid=(N // BLK,),
        in_specs=[_spec_z, _spec_z, _spec_dp, _spec_w, _spec_b],
        out_specs=_spec_z,
        out_shape=jax.ShapeDtypeStruct((NC, N, H), jnp.float32),
    )(s, z, deg_part, W, b)


def _tc_fin(s, z, deg_part, b):
    return pl.pallas_call(
        _fin_body,
        grid=(N // BLK,),
        in_specs=[_spec_z, _spec_z, _spec_dp, _spec_b],
        out_specs=pl.BlockSpec((BLK, D), lambda i: (i, 0)),
        out_shape=jax.ShapeDtypeStruct((N, D), jnp.float32),
    )(s, z, deg_part, b)


def kernel(x, edge_index, W1, b1, W2, b2, W3, b3):
    r_arr = edge_index[0].reshape(NW, 1, EPT)
    c_arr = edge_index[1].reshape(NW, 1, EPT)
    rlo, rhi, co = _tc_remap(r_arr, c_arr)
    ridx = jnp.stack([rlo, rhi]).reshape(NC, NW, NCHUNK, K)
    cidx = co.reshape(NW, NCHUNK, K)

    deg_part = _sc_deg(cidx).reshape(NC, NPAD).T

    b1r, b2r, b3r = (b.reshape(1, D) for b in (b1, b2, b3))

    z1 = _tc_k1(x, W1, deg_part)
    s1 = _sc_agg(z1.reshape(NC * N, H), ridx, cidx)
    z2 = _tc_mid(s1, z1, deg_part, W2, b1r)
    s2 = _sc_agg(z2.reshape(NC * N, H), ridx, cidx)
    z3 = _tc_mid(s2, z2, deg_part, W3, b2r)
    s3 = _sc_agg(z3.reshape(NC * N, H), ridx, cidx)
    return _tc_fin(s3, z3, deg_part, b3r)


# speedup vs baseline: 8.7877x; 1.0678x over previous
import functools

import jax
import jax.numpy as jnp
from jax import lax
from jax.experimental import pallas as pl
from jax.experimental.pallas import tpu as pltpu
from jax.experimental.pallas import tpu_sc as plsc

N = 10000
E = 160000
D = 256
H = 128
NC, NS = 2, 16
NW = NC * NS
EPT = E // NW
K = 64
NCHUNK = 80
EPT_PAD = NCHUNK * K
NPAD = 10240
RPT = NPAD // NS
NBUF = 4
DUMMY = N
BLK = 1000

_mesh = plsc.VectorSubcoreMesh(
    core_axis_name="c", subcore_axis_name="s", num_cores=NC, num_subcores=NS
)


def _remap_body(r_ref, c_ref, rlo_ref, rhi_ref, co_ref):
    rr = r_ref[...]
    cc = c_ref[...]
    is_self = rr == cc
    co = jnp.where(is_self, DUMMY, cc)
    pad = ((0, 0), (0, 0), (0, EPT_PAD - EPT))
    rp = jnp.pad(rr, pad, constant_values=0)
    rlo_ref[...] = rp
    rhi_ref[...] = rp + N
    co_ref[...] = jnp.pad(co, pad, constant_values=DUMMY)


def _tc_remap(r_arr, c_arr):
    return pl.pallas_call(
        _remap_body,
        grid=(NW,),
        in_specs=[
            pl.BlockSpec((1, 1, EPT), lambda w: (w, 0, 0)),
            pl.BlockSpec((1, 1, EPT), lambda w: (w, 0, 0)),
        ],
        out_specs=[
            pl.BlockSpec((1, 1, EPT_PAD), lambda w: (w, 0, 0)),
            pl.BlockSpec((1, 1, EPT_PAD), lambda w: (w, 0, 0)),
            pl.BlockSpec((1, 1, EPT_PAD), lambda w: (w, 0, 0)),
        ],
        out_shape=[
            jax.ShapeDtypeStruct((NW, 1, EPT_PAD), jnp.int32),
            jax.ShapeDtypeStruct((NW, 1, EPT_PAD), jnp.int32),
            jax.ShapeDtypeStruct((NW, 1, EPT_PAD), jnp.int32),
        ],
    )(r_arr, c_arr)


@functools.partial(
    pl.kernel,
    out_type=jax.ShapeDtypeStruct((NC * NPAD,), jnp.float32),
    mesh=_mesh,
    scratch_types=[
        pltpu.VMEM((NCHUNK, K), jnp.int32),
        pltpu.VMEM((K,), jnp.float32),
        pltpu.VMEM((RPT,), jnp.float32),
        pltpu.VMEM_SHARED((NPAD,), jnp.float32),
    ],
)
def _sc_deg(cidx_hbm, deg_out, cv, ones_v, zb, deg_sp):
    cid = lax.axis_index("c")
    sid = lax.axis_index("s")
    wid = cid * NS + sid
    one16 = jnp.full((16,), 1.0, jnp.float32)
    zero16 = jnp.zeros((16,), jnp.float32)
    for i in range(K // 16):
        ones_v[pl.ds(i * 16, 16)] = one16

    def zrow(i, _):
        zb[pl.ds(i * 16, 16)] = zero16
        return 0

    lax.fori_loop(0, RPT // 16, zrow, 0)
    pltpu.sync_copy(zb, deg_sp.at[pl.ds(sid * RPT, RPT)])
    plsc.subcore_barrier()
    pltpu.sync_copy(cidx_hbm.at[wid], cv)

    def body(j, _):
        pltpu.sync_copy(ones_v, deg_sp.at[cv.at[j]], add=True)
        return 0

    lax.fori_loop(0, NCHUNK, body, 0)
    plsc.subcore_barrier()
    pltpu.sync_copy(deg_sp.at[pl.ds(sid * RPT, RPT)], zb)
    pltpu.sync_copy(zb, deg_out.at[pl.ds(cid * NPAD + sid * RPT, RPT)])


@functools.partial(
    pl.kernel,
    out_type=jax.ShapeDtypeStruct((NC, NPAD, H), jnp.float32),
    mesh=_mesh,
    scratch_types=[
        pltpu.VMEM((NCHUNK // 2, K), jnp.int32),
        pltpu.VMEM((NCHUNK // 2, K), jnp.int32),
        pltpu.VMEM((K, H), jnp.float32),
        pltpu.VMEM((K, H), jnp.float32),
        pltpu.VMEM((K, H), jnp.float32),
        pltpu.VMEM((K, H), jnp.float32),
        pltpu.VMEM_SHARED((NPAD, H), jnp.float32),
        pltpu.SemaphoreType.DMA,
        pltpu.SemaphoreType.DMA,
        pltpu.SemaphoreType.DMA,
        pltpu.SemaphoreType.DMA,
        pltpu.SemaphoreType.DMA,
        pltpu.SemaphoreType.DMA,
        pltpu.SemaphoreType.DMA,
        pltpu.SemaphoreType.DMA,
    ],
)
def _sc_agg(z_hbm, ridx_hbm, cidx_hbm, s_hbm, rv, cv, g0, g1, g2, g3,
            acc, gs0, gs1, gs2, gs3, ss0, ss1, ss2, ss3):
    cid = lax.axis_index("c")
    sid = lax.axis_index("s")
    gb = (g0, g1, g2, g3)
    gs = (gs0, gs1, gs2, gs3)
    ss = (ss0, ss1, ss2, ss3)
    zero16 = jnp.zeros((16,), jnp.float32)

    def zrow(i, _):
        for k in range(H // 16):
            g0[i, pl.ds(k * 16, 16)] = zero16
        return 0

    lax.fori_loop(0, K, zrow, 0)
    zdesc = [
        pltpu.async_copy(g0, acc.at[pl.ds(sid * RPT + q * K, K)], ss0)
        for q in range(RPT // K)
    ]
    for d in zdesc:
        d.wait()
    plsc.subcore_barrier()

    HC = NCHUNK // 2
    for part in range(4):
        slab = sid + (part // 2) * NS
        half = part % 2
        pltpu.sync_copy(ridx_hbm.at[cid, slab, pl.ds(half * HC, HC)], rv)
        pltpu.sync_copy(cidx_hbm.at[slab, pl.ds(half * HC, HC)], cv)
        for b in range(NBUF):
            pltpu.async_copy(z_hbm.at[rv.at[b]], gb[b], gs[b])

        def body(t, _):
            for b in range(NBUF):
                j = t * NBUF + b
                pltpu.make_async_copy(z_hbm.at[pl.ds(0, K)], gb[b], gs[b]).wait()
                pltpu.async_copy(gb[b], acc.at[cv.at[j]], ss[b], add=True)
            for b in range(NBUF):
                jn = (t + 1) * NBUF + b
                pltpu.make_async_copy(z_hbm.at[pl.ds(0, K)], gb[b], ss[b]).wait()

                @pl.when(jn < HC)
                def _():
                    pltpu.async_copy(z_hbm.at[rv.at[jn]], gb[b], gs[b])

            return 0

        lax.fori_loop(0, HC // NBUF, body, 0)
    plsc.subcore_barrier()
    wdesc = [None] * NBUF
    for q in range(RPT // K):
        b = q % NBUF
        if wdesc[b] is not None:
            wdesc[b].wait()
        pltpu.sync_copy(acc.at[pl.ds(sid * RPT + q * K, K)], gb[b])
        wdesc[b] = pltpu.async_copy(
            gb[b], s_hbm.at[cid, pl.ds(sid * RPT + q * K, K)], gs[b]
        )
    for d in wdesc:
        if d is not None:
            d.wait()


def _dinv(dp_ref):
    deg = dp_ref[:, 0] + dp_ref[:, 1] + 1.0
    return lax.rsqrt(deg)


def _k1_body(x_ref, w_ref, dp_ref, z_ref):
    dinv = _dinv(dp_ref)
    y = jnp.dot(x_ref[...], w_ref[...], preferred_element_type=jnp.float32)
    z = y * dinv[:, None]
    z_ref[0] = z[:, :H]
    z_ref[1] = z[:, H:]


def _mid_body(s_ref, z_ref, dp_ref, w_ref, b_ref, zo_ref):
    dinv = _dinv(dp_ref)
    t = jnp.concatenate([s_ref[0] + z_ref[0], s_ref[1] + z_ref[1]], axis=1)
    h = jnp.maximum(t * dinv[:, None] + b_ref[...], 0.0)
    y = jnp.dot(h, w_ref[...], preferred_element_type=jnp.float32)
    zn = y * dinv[:, None]
    zo_ref[0] = zn[:, :H]
    zo_ref[1] = zn[:, H:]


def _fin_body(s_ref, z_ref, dp_ref, b_ref, o_ref):
    dinv = _dinv(dp_ref)
    t = jnp.concatenate([s_ref[0] + z_ref[0], s_ref[1] + z_ref[1]], axis=1)
    o_ref[...] = t * dinv[:, None] + b_ref[...]


_spec_z = pl.BlockSpec((NC, BLK, H), lambda i: (0, i, 0))
_spec_dp = pl.BlockSpec((BLK, NC), lambda i: (i, 0))
_spec_w = pl.BlockSpec((D, D), lambda i: (0, 0))
_spec_b = pl.BlockSpec((1, D), lambda i: (0, 0))


def _tc_k1(x, W, deg_part):
    return pl.pallas_call(
        _k1_body,
        grid=(N // BLK,),
        in_specs=[
            pl.BlockSpec((BLK, D), lambda i: (i, 0)),
            _spec_w,
            _spec_dp,
        ],
        out_specs=_spec_z,
        out_shape=jax.ShapeDtypeStruct((NC, N, H), jnp.float32),
    )(x, W, deg_part)


def _tc_mid(s, z, deg_part, W, b):
    return pl.pallas_call(
        _mid_body,
        grid=(N // BLK,),
        in_specs=[_spec_z, _spec_z, _spec_dp, _spec_w, _spec_b],
        out_specs=_spec_z,
        out_shape=jax.ShapeDtypeStruct((NC, N, H), jnp.float32),
    )(s, z, deg_part, W, b)


def _tc_fin(s, z, deg_part, b):
    return pl.pallas_call(
        _fin_body,
        grid=(N // BLK,),
        in_specs=[_spec_z, _spec_z, _spec_dp, _spec_b],
        out_specs=pl.BlockSpec((BLK, D), lambda i: (i, 0)),
        out_shape=jax.ShapeDtypeStruct((N, D), jnp.float32),
    )(s, z, deg_part, b)


def kernel(x, edge_index, W1, b1, W2, b2, W3, b3):
    r_arr = edge_index[0].reshape(NW, 1, EPT)
    c_arr = edge_index[1].reshape(NW, 1, EPT)
    rlo, rhi, co = _tc_remap(r_arr, c_arr)
    ridx = jnp.stack([rlo, rhi]).reshape(NC, NW, NCHUNK, K)
    cidx = co.reshape(NW, NCHUNK, K)

    deg_part = _sc_deg(cidx).reshape(NC, NPAD).T

    b1r, b2r, b3r = (b.reshape(1, D) for b in (b1, b2, b3))

    z1 = _tc_k1(x, W1, deg_part)
    s1 = _sc_agg(z1.reshape(NC * N, H), ridx, cidx)
    z2 = _tc_mid(s1, z1, deg_part, W2, b1r)
    s2 = _sc_agg(z2.reshape(NC * N, H), ridx, cidx)
    z3 = _tc_mid(s2, z2, deg_part, W3, b2r)
    s3 = _sc_agg(z3.reshape(NC * N, H), ridx, cidx)
    return _tc_fin(s3, z3, deg_part, b3r)
